# Initial kernel scaffold; baseline (speedup 1.0000x reference)
import jax
import jax.numpy as jnp
from jax.experimental import pallas as pl


def kernel(xyz, feat, W_feat, W_dir, gamma):
    raise NotImplementedError("write your pallas kernel here")


# speedup vs baseline: 334.1345x; 334.1345x over previous
import functools

import jax
import jax.numpy as jnp
from jax import lax
from jax.experimental import pallas as pl
from jax.experimental.pallas import tpu as pltpu
from jax.experimental.pallas import tpu_sc as plsc

EPS = 1e-6
NPOINT = 512
K = 24
NEG_SLOPE = 0.1
N = 2048
B = 4
C = 64
COUT = 128
D_TAB = 256
M_BLK = 128
NS_ITERS = 24


def _fps_body(xt_ref, pos_n_ref, pos_m_ref, out_ref):
    X = xt_ref[:, 0, :]
    Y = xt_ref[:, 1, :]
    Z = xt_ref[:, 2, :]
    iota_n = pos_n_ref[...]
    iota_m = pos_m_ref[...]

    def body(i, st):
        distance, far, cent = st
        cent = jnp.where(iota_m == i, far, cent)
        sel = iota_n == far
        cx = jnp.sum(jnp.where(sel, X, 0.0), axis=1, keepdims=True)
        cy = jnp.sum(jnp.where(sel, Y, 0.0), axis=1, keepdims=True)
        cz = jnp.sum(jnp.where(sel, Z, 0.0), axis=1, keepdims=True)
        dx = X - cx
        dy = Y - cy
        dz = Z - cz
        dist = dx * dx + dy * dy + dz * dz
        distance = jnp.minimum(distance, dist)
        m = jnp.max(distance, axis=1, keepdims=True)
        cols = jnp.where(distance == m, iota_n, N)
        far = jnp.min(cols, axis=1, keepdims=True)
        return (distance, far, cent)

    distance0 = X * 0.0 + 1e10
    far0 = jnp.zeros((B, 1), dtype=jnp.int32)
    cent0 = iota_m * 0
    _, _, cent = lax.fori_loop(0, NPOINT, body, (distance0, far0, cent0))
    out_ref[...] = cent


def _fps(xyzT):
    pos_n = jnp.broadcast_to(jnp.arange(N, dtype=jnp.int32), (B, N))
    pos_m = jnp.broadcast_to(jnp.arange(NPOINT, dtype=jnp.int32), (B, NPOINT))
    return pl.pallas_call(
        _fps_body,
        out_shape=jax.ShapeDtypeStruct((B, NPOINT), jnp.int32),
    )(xyzT, pos_n, pos_m)


def _knn_body(xt_ref, fps_ref, pos_ref, idx_ref, xyz_out_ref):
    X = xt_ref[0, 0:1, :]
    Y = xt_ref[0, 1:2, :]
    Z = xt_ref[0, 2:3, :]
    fidx = fps_ref[0]
    iota_n = pos_ref[...]
    sel = iota_n == fidx
    cx = jnp.sum(jnp.where(sel, X, 0.0), axis=1, keepdims=True)
    cy = jnp.sum(jnp.where(sel, Y, 0.0), axis=1, keepdims=True)
    cz = jnp.sum(jnp.where(sel, Z, 0.0), axis=1, keepdims=True)
    xyz_out_ref[0] = jnp.concatenate([cx, cy, cz], axis=1)

    dx = cx - X
    dy = cy - Y
    dz = cz - Z
    d2 = dx * dx + dy * dy + dz * dz
    picks = []
    for _ in range(K):
        m = jnp.min(d2, axis=1, keepdims=True)
        cols = jnp.where(d2 == m, iota_n, N)
        pick = jnp.min(cols, axis=1, keepdims=True)
        picks.append(pick)
        d2 = jnp.where(iota_n == pick, 3.0e38, d2)
    idx_ref[0] = jnp.concatenate(picks, axis=1)


def _knn(xyzT, fps3):
    n_mb = NPOINT // M_BLK
    call = pl.pallas_call(
        _knn_body,
        grid=(B, n_mb),
        in_specs=[
            pl.BlockSpec((1, 3, N), lambda b, m: (b, 0, 0)),
            pl.BlockSpec((1, M_BLK, 1), lambda b, m: (b, m, 0)),
            pl.BlockSpec((M_BLK, N), lambda b, m: (0, 0)),
        ],
        out_specs=[
            pl.BlockSpec((1, M_BLK, K), lambda b, m: (b, m, 0)),
            pl.BlockSpec((1, M_BLK, 3), lambda b, m: (b, m, 0)),
        ],
        out_shape=[
            jax.ShapeDtypeStruct((B, NPOINT, K), jnp.int32),
            jax.ShapeDtypeStruct((B, NPOINT, 3), jnp.float32),
        ],
    )
    pos = jnp.broadcast_to(jnp.arange(N, dtype=jnp.int32), (M_BLK, N))
    return call(xyzT, fps3, pos)


def _sc_gather(table, idx_all):
    n_rows = idx_all.shape[0]
    info = plsc.get_sparse_core_info()
    nw = info.num_cores * info.num_subcores
    b_per_w = n_rows // nw
    chunk = 400
    n_chunks = b_per_w // chunk
    mesh = plsc.VectorSubcoreMesh(core_axis_name="c", subcore_axis_name="s")

    @functools.partial(
        pl.kernel,
        mesh=mesh,
        out_type=jax.ShapeDtypeStruct((n_rows, D_TAB), jnp.float32),
        scratch_types=[
            pltpu.VMEM((chunk,), jnp.int32),
            pltpu.VMEM((chunk, D_TAB), jnp.float32),
            pltpu.SemaphoreType.DMA,
        ],
    )
    def gk(table_hbm, idx_hbm, out_hbm, idx_v, rows_v, sem):
        wid = lax.axis_index("s") * info.num_cores + lax.axis_index("c")
        base = wid * b_per_w
        for j in range(n_chunks):
            off = base + j * chunk
            pltpu.sync_copy(idx_hbm.at[pl.ds(off, chunk)], idx_v)
            pltpu.async_copy(table_hbm.at[idx_v], rows_v, sem).wait()
            pltpu.sync_copy(rows_v, out_hbm.at[pl.ds(off, chunk)])

    return gk(table, idx_all)


def _edge_body(g_ref, a_ref, wnf_ref, waf_ref, wxf_ref, wnd_ref, wad_ref,
               wxd_ref, out_ref):
    g = g_ref[0]
    a = a_ref[0]
    R = M_BLK * K

    def rep(x):
        w = x.shape[1]
        return jnp.broadcast_to(x[:, None, :], (M_BLK, K, w)).reshape(R, w)

    ps = []
    ds = []
    for v in range(3):
        Gv = g[:, v * C:(v + 1) * C]
        Av = a[:, v * C:(v + 1) * C]
        relv = g[:, 192 + v:193 + v] - rep(a[:, 192 + v:193 + v])
        pv = (jnp.dot(Gv, wnf_ref[...], preferred_element_type=jnp.float32)
              + rep(jnp.dot(Av, waf_ref[...], preferred_element_type=jnp.float32))
              + relv * wxf_ref[...])
        dv = (jnp.dot(Gv, wnd_ref[...], preferred_element_type=jnp.float32)
              + rep(jnp.dot(Av, wad_ref[...], preferred_element_type=jnp.float32))
              + relv * wxd_ref[...])
        ps.append(pv)
        ds.append(dv)

    dotpd = ps[0] * ds[0] + ps[1] * ds[1] + ps[2] * ds[2]
    dns = ds[0] * ds[0] + ds[1] * ds[1] + ds[2] * ds[2]
    mask = (dotpd >= 0).astype(jnp.float32)
    coef = dotpd / (dns + EPS)
    for v in range(3):
        h = (NEG_SLOPE * ps[v]
             + (1 - NEG_SLOPE) * (mask * ps[v]
                                  + (1 - mask) * (ps[v] - coef * ds[v])))
        out_ref[0, v] = jnp.mean(h.reshape(M_BLK, K, COUT), axis=1)


def _edge_mlp(g, a, wnf, waf, wxf, wnd, wad, wxd):
    n_mb = NPOINT // M_BLK
    return pl.pallas_call(
        _edge_body,
        grid=(B, n_mb),
        in_specs=[
            pl.BlockSpec((1, M_BLK * K, D_TAB), lambda b, m: (b, m, 0)),
            pl.BlockSpec((1, M_BLK, D_TAB), lambda b, m: (b, m, 0)),
            pl.BlockSpec((C, COUT), lambda b, m: (0, 0)),
            pl.BlockSpec((C, COUT), lambda b, m: (0, 0)),
            pl.BlockSpec((1, COUT), lambda b, m: (0, 0)),
            pl.BlockSpec((C, COUT), lambda b, m: (0, 0)),
            pl.BlockSpec((C, COUT), lambda b, m: (0, 0)),
            pl.BlockSpec((1, COUT), lambda b, m: (0, 0)),
        ],
        out_specs=pl.BlockSpec((1, 3, M_BLK, COUT), lambda b, m: (b, 0, m, 0)),
        out_shape=jax.ShapeDtypeStruct((B, 3, NPOINT, COUT), jnp.float32),
    )(g, a, wnf, waf, wxf, wnd, wad, wxd)


def _mm3(a, b):
    return (a[:, 0:1] * b[0:1, :] + a[:, 1:2] * b[1:2, :]
            + a[:, 2:3] * b[2:3, :])


def _whiten_body(x_ref, g_ref, out_ref):
    x = x_ref[0]
    MT = x.shape[1]
    mu = jnp.mean(x, axis=1, keepdims=True)
    xc = x - mu
    x0 = xc[0:1, :]
    x1 = xc[1:2, :]
    x2 = xc[2:3, :]
    denom = float(MT) + EPS
    c00 = jnp.sum(x0 * x0) / denom + 1e-5
    c11 = jnp.sum(x1 * x1) / denom + 1e-5
    c22 = jnp.sum(x2 * x2) / denom + 1e-5
    c01 = jnp.sum(x0 * x1) / denom
    c02 = jnp.sum(x0 * x2) / denom
    c12 = jnp.sum(x1 * x2) / denom
    r3 = lax.broadcasted_iota(jnp.int32, (3, 3), 0)
    c3 = lax.broadcasted_iota(jnp.int32, (3, 3), 1)

    def E(i, j):
        return ((r3 == i) & (c3 == j)).astype(jnp.float32)

    eye = E(0, 0) + E(1, 1) + E(2, 2)
    cov = (c00 * E(0, 0) + c11 * E(1, 1) + c22 * E(2, 2)
           + c01 * (E(0, 1) + E(1, 0)) + c02 * (E(0, 2) + E(2, 0))
           + c12 * (E(1, 2) + E(2, 1)))
    s = c00 + c11 + c22
    A = cov * (1.0 / s)

    def ns(i, st):
        Yk, Zk = st
        T = 3.0 * eye - _mm3(Zk, Yk)
        return (0.5 * _mm3(Yk, T), 0.5 * _mm3(T, Zk))

    _, Zf = lax.fori_loop(0, NS_ITERS, ns, (A, eye))
    Wz = Zf * lax.rsqrt(s)
    w00 = jnp.sum(Wz * E(0, 0))
    w01 = jnp.sum(Wz * E(0, 1))
    w02 = jnp.sum(Wz * E(0, 2))
    w10 = jnp.sum(Wz * E(1, 0))
    w11 = jnp.sum(Wz * E(1, 1))
    w12 = jnp.sum(Wz * E(1, 2))
    w20 = jnp.sum(Wz * E(2, 0))
    w21 = jnp.sum(Wz * E(2, 1))
    w22 = jnp.sum(Wz * E(2, 2))
    y0 = w00 * x0 + w01 * x1 + w02 * x2
    y1 = w10 * x0 + w11 * x1 + w12 * x2
    y2 = w20 * x0 + w21 * x1 + w22 * x2
    gm = g_ref[...]
    out_ref[0] = jnp.concatenate([y0 * gm, y1 * gm, y2 * gm], axis=0)


def _whiten(xf, gamma_rep):
    MT = xf.shape[2]
    return pl.pallas_call(
        _whiten_body,
        grid=(B,),
        in_specs=[
            pl.BlockSpec((1, 3, MT), lambda b: (b, 0, 0)),
            pl.BlockSpec((1, MT), lambda b: (0, 0)),
        ],
        out_specs=pl.BlockSpec((1, 3, MT), lambda b: (b, 0, 0)),
        out_shape=jax.ShapeDtypeStruct((B, 3, MT), jnp.float32),
    )(xf, gamma_rep)


def kernel(xyz, feat, W_feat, W_dir, gamma):
    xyzT = jnp.transpose(xyz, (0, 2, 1))
    fps_idx = _fps(xyzT)
    knn_idx, new_xyz = _knn(xyzT, fps_idx[:, :, None])

    featP = jnp.transpose(feat, (0, 3, 2, 1)).reshape(B, N, 3 * C)
    table = jnp.concatenate(
        [featP, xyz, jnp.zeros((B, N, D_TAB - 3 * C - 3), jnp.float32)],
        axis=2).reshape(B * N, D_TAB)
    offs = (jnp.arange(B, dtype=jnp.int32) * N)[:, None]
    neigh_g = (knn_idx.reshape(B, -1) + offs).reshape(-1)
    anch_g = (fps_idx + offs).reshape(-1)
    idx_all = jnp.concatenate([neigh_g, anch_g])
    rows = _sc_gather(table, idx_all)
    n_neigh = B * NPOINT * K
    g = rows[:n_neigh].reshape(B, NPOINT * K, D_TAB)
    a = rows[n_neigh:].reshape(B, NPOINT, D_TAB)

    wnf = jnp.transpose(W_feat[:, :C])
    waf = jnp.transpose(W_feat[:, C:2 * C] - W_feat[:, :C])
    wxf = W_feat[:, 2 * C].reshape(1, COUT)
    wnd = jnp.transpose(W_dir[:, :C])
    wad = jnp.transpose(W_dir[:, C:2 * C] - W_dir[:, :C])
    wxd = W_dir[:, 2 * C].reshape(1, COUT)
    pooled = _edge_mlp(g, a, wnf, waf, wxf, wnd, wad, wxd)

    xf = jnp.transpose(pooled, (0, 1, 3, 2)).reshape(B, 3, COUT * NPOINT)
    gamma_rep = jnp.repeat(gamma.reshape(-1), NPOINT).reshape(1, COUT * NPOINT)
    xw = _whiten(xf, gamma_rep)
    out_feat = xw.reshape(B, 3, COUT, NPOINT).transpose(0, 2, 1, 3)
    return new_xyz, out_feat
